# baseline (speedup 1.0000x reference)
import jax
import jax.numpy as jnp
from jax import lax
from jax.experimental import pallas as pl
from jax.experimental.pallas import tpu as pltpu
from jax.experimental.pallas import tpu_sc as plsc

NC = 2
NS = 16
NW = NC * NS
L = 16

CHUNK = 1024


def _emb_body(tokens_hbm, table_hbm, out_hbm, idx_v, rows_v, sem):
    D = table_hbm.shape[1]
    scale = float(D) ** 0.5
    B = tokens_hbm.shape[0]
    b_per_w = B // NW
    n_chunks = b_per_w // CHUNK
    wid = lax.axis_index("s") * NC + lax.axis_index("c")
    base0 = wid * b_per_w

    def chunk_body(k, carry):
        base = base0 + k * CHUNK
        pltpu.sync_copy(tokens_hbm.at[pl.ds(base, CHUNK)], idx_v)
        pltpu.async_copy(table_hbm.at[idx_v], rows_v, sem).wait()

        def scale_row(r, c):
            for j in range(D // L):
                sl = pl.ds(j * L, L)
                rows_v[r, sl] = rows_v[r, sl] * scale
            return c

        lax.fori_loop(0, CHUNK, scale_row, 0)
        pltpu.sync_copy(rows_v, out_hbm.at[pl.ds(base, CHUNK)])
        return carry

    lax.fori_loop(0, n_chunks, chunk_body, 0)


def kernel(tokens, table):
    B = tokens.size
    D = table.shape[1]
    toks = tokens.reshape(B).astype(jnp.int32)
    mesh = plsc.VectorSubcoreMesh(
        core_axis_name="c", subcore_axis_name="s",
        num_cores=NC, num_subcores=NS,
    )
    out = pl.kernel(
        _emb_body,
        out_type=jax.ShapeDtypeStruct((B, D), table.dtype),
        mesh=mesh,
        scratch_types=[
            pltpu.VMEM((CHUNK,), jnp.int32),
            pltpu.VMEM((CHUNK, D), jnp.float32),
            pltpu.SemaphoreType.DMA,
        ],
        compiler_params=pltpu.CompilerParams(use_tc_tiling_on_sc=False),
    )(toks, table)
    return out.reshape(*tokens.shape, D)


# speedup vs baseline: 1.1080x; 1.1080x over previous
import jax
import jax.numpy as jnp
from jax import lax
from jax.experimental import pallas as pl
from jax.experimental.pallas import tpu as pltpu
from jax.experimental.pallas import tpu_sc as plsc

NC = 2
NS = 16
NW = NC * NS
L = 16

NBUF = 4
CHUNK = 400
PREF = 2


def _emb_body(tokens_hbm, table_hbm, out_hbm, idx_v, rows, gsems, wsems):
    D = table_hbm.shape[1]
    scale = float(D) ** 0.5
    B = tokens_hbm.shape[0]
    b_per_w = B // NW
    n_chunks = b_per_w // CHUNK
    wid = lax.axis_index("s") * NC + lax.axis_index("c")
    base0 = wid * b_per_w

    def start_gather(k, b):
        pltpu.sync_copy(tokens_hbm.at[pl.ds(base0 + k * CHUNK, CHUNK)],
                        idx_v.at[b])
        pltpu.async_copy(table_hbm.at[idx_v.at[b]], rows[b], gsems[b])

    for j in range(PREF):
        start_gather(j, j)

    @pl.loop(0, n_chunks, step=NBUF)
    def block(k0):
        for b in range(NBUF):
            k = k0 + b
            q = k + PREF
            qb = (b + PREF) % NBUF

            @pl.when(q < n_chunks)
            def _():
                @pl.when(k >= PREF)
                def _():
                    pltpu.make_async_copy(
                        rows[qb], out_hbm.at[pl.ds(base0, CHUNK)], wsems[qb]
                    ).wait()
                start_gather(q, qb)

            pltpu.make_async_copy(
                table_hbm.at[idx_v.at[b]], rows[b], gsems[b]
            ).wait()

            @plsc.parallel_loop(0, CHUNK, step=1, unroll=8)
            def scale_row(r):
                for j in range(D // L):
                    sl = pl.ds(j * L, L)
                    rows[b][r, sl] = rows[b][r, sl] * scale

            pltpu.async_copy(rows[b],
                             out_hbm.at[pl.ds(base0 + k * CHUNK, CHUNK)],
                             wsems[b])

    for b in range(NBUF):
        pltpu.make_async_copy(
            rows[b], out_hbm.at[pl.ds(base0, CHUNK)], wsems[b]
        ).wait()


def kernel(tokens, table):
    B = tokens.size
    D = table.shape[1]
    toks = tokens.reshape(B).astype(jnp.int32)
    mesh = plsc.VectorSubcoreMesh(
        core_axis_name="c", subcore_axis_name="s",
        num_cores=NC, num_subcores=NS,
    )
    out = pl.kernel(
        _emb_body,
        out_type=jax.ShapeDtypeStruct((B, D), table.dtype),
        mesh=mesh,
        scratch_types=[
            pltpu.VMEM((NBUF, CHUNK), jnp.int32),
            [pltpu.VMEM((CHUNK, D), jnp.float32) for _ in range(NBUF)],
            [pltpu.SemaphoreType.DMA for _ in range(NBUF)],
            [pltpu.SemaphoreType.DMA for _ in range(NBUF)],
        ],
        compiler_params=pltpu.CompilerParams(use_tc_tiling_on_sc=False),
    )(toks, table)
    return out.reshape(*tokens.shape, D)
